# Initial kernel scaffold; baseline (speedup 1.0000x reference)
import jax
import jax.numpy as jnp
from jax.experimental import pallas as pl


def kernel(points, normals):
    raise NotImplementedError("write your pallas kernel here")


# speedup vs baseline: 4.6604x; 4.6604x over previous
import jax
import jax.numpy as jnp
from jax.experimental import pallas as pl
from jax.experimental.pallas import tpu as pltpu

KNN_K = 33
SIGMA = 0.75
EPS = 1e-10
TILE = 128


def _eps_denom(x):
    return jnp.where(x < EPS, EPS, x)


def _noisy_d2(rowp_ref, rowbf_ref, allbf_ref, pts_ref):
    px = pts_ref[0, :][None, :]
    py = pts_ref[1, :][None, :]
    pz = pts_ref[2, :][None, :]
    rx = rowp_ref[0, :][:, None]
    ry = rowp_ref[1, :][:, None]
    rz = rowp_ref[2, :][:, None]
    sq_all = px * px + py * py + pz * pz
    sq_row = rx * rx + ry * ry + rz * rz
    cross = jax.lax.dot_general(
        rowbf_ref[...], allbf_ref[...],
        (((1,), (0,)), ((), ())),
        preferred_element_type=jnp.float32)
    return jnp.maximum(sq_row + sq_all - 2.0 * cross, 0.0)


def _exact_d2(rowp_ref, pts_ref):
    dx = rowp_ref[0, :][:, None] - pts_ref[0, :][None, :]
    dy = rowp_ref[1, :][:, None] - pts_ref[1, :][None, :]
    dz = rowp_ref[2, :][:, None] - pts_ref[2, :][None, :]
    return dx * dx + dy * dy + dz * dz


def _stage1_kernel(pts_ref, nrm_ref, rowp_ref, rowbf_ref, allbf_ref,
                   nd_ref, h_ref, mask_ref, cur_ref, kept_ref):
    d2 = _noisy_d2(rowp_ref, rowbf_ref, allbf_ref, pts_ref)
    t, p = d2.shape
    col = jax.lax.broadcasted_iota(jnp.int32, (t, p), 1)
    cur_ref[...] = d2
    kept_ref[...] = jnp.zeros((t, p), jnp.float32)

    def body(k, _):
        cur = cur_ref[...]
        m = jnp.min(cur, axis=1, keepdims=True)
        tie = cur == m
        isel = jnp.min(jnp.where(tie, col, p), axis=1, keepdims=True)
        rem = tie & (col == isel)
        cur_ref[...] = jnp.where(rem, jnp.inf, cur)
        kept_ref[...] = jnp.where(rem & (k > 0), 1.0, kept_ref[...])
        return 0

    jax.lax.fori_loop(0, KNN_K, body, 0)
    kept = kept_ref[...] != 0.0

    d = _exact_d2(rowp_ref, pts_ref)
    dm = jnp.where(kept, d, 0.0)
    h = jnp.sum(dm, axis=1, keepdims=True) * (4.0 / (KNN_K - 1.0))
    w = jnp.maximum(1.0 - d / _eps_denom(h), 0.0)
    w = w * w
    phi = jnp.where(kept, w * w, 0.0)
    den = _eps_denom(jnp.sum(phi, axis=1, keepdims=True))
    nx = nrm_ref[0, :][None, :]
    ny = nrm_ref[1, :][None, :]
    nz = nrm_ref[2, :][None, :]
    nd_ref[0, :] = jnp.sum(phi * nx, axis=1) / den[:, 0]
    nd_ref[1, :] = jnp.sum(phi * ny, axis=1) / den[:, 0]
    nd_ref[2, :] = jnp.sum(phi * nz, axis=1) / den[:, 0]
    h_ref[:] = h[:, 0]
    mask_ref[...] = kept.astype(jnp.int8)


def _stage2_kernel(pts_ref, nd_ref, rowp_ref, rownd_ref, h_ref, mask_ref,
                   loss_ref):
    kept = mask_ref[...] != 0
    d = _exact_d2(rowp_ref, pts_ref)
    h = h_ref[:][:, None]
    w = jnp.maximum(1.0 - d / _eps_denom(h), 0.0)
    w = w * w
    phi = jnp.where(kept, w * w, 0.0)

    ndx = nd_ref[0, :][None, :]
    ndy = nd_ref[1, :][None, :]
    ndz = nd_ref[2, :][None, :]
    inv_all = 1.0 / jnp.maximum(jnp.sqrt(ndx * ndx + ndy * ndy + ndz * ndz),
                                1e-12)
    ux, uy, uz = ndx * inv_all, ndy * inv_all, ndz * inv_all
    s_all = ux * ux + uy * uy + uz * uz

    rdx = rownd_ref[0, :][:, None]
    rdy = rownd_ref[1, :][:, None]
    rdz = rownd_ref[2, :][:, None]
    inv_row = 1.0 / jnp.maximum(jnp.sqrt(rdx * rdx + rdy * rdy + rdz * rdz),
                                1e-12)
    vx, vy, vz = rdx * inv_row, rdy * inv_row, rdz * inv_row
    s_row = vx * vx + vy * vy + vz * vz

    dot = vx * ux + vy * uy + vz * uz
    inv_sig = 1.0 / (SIGMA * SIGMA)
    normal_w = jnp.exp(-(s_row + s_all - 2.0 * dot) * inv_sig)
    w2 = phi * normal_w

    px = pts_ref[0, :][None, :]
    py = pts_ref[1, :][None, :]
    pz = pts_ref[2, :][None, :]
    rx = rowp_ref[0, :][:, None]
    ry = rowp_ref[1, :][:, None]
    rz = rowp_ref[2, :][:, None]
    a_all = px * ndx + py * ndy + pz * ndz
    pdot = rx * ndx + ry * ndy + rz * ndz
    inner = pdot - a_all

    num = jnp.sum(inner * w2, axis=1)
    den = _eps_denom(jnp.sum(w2, axis=1))
    dist = num / den
    loss_ref[:] = dist * dist


@jax.jit
def _run(points, normals):
    pts = points[0].T.astype(jnp.float32)
    nrm = normals[0].T.astype(jnp.float32)
    pts_rows_bf = points[0].astype(jnp.bfloat16)
    pts_all_bf = pts.astype(jnp.bfloat16)
    p = pts.shape[1]
    grid = (p // TILE,)
    full = pl.BlockSpec((3, p), lambda i: (0, 0))
    rowb = pl.BlockSpec((3, TILE), lambda i: (0, i))
    vecb = pl.BlockSpec((TILE,), lambda i: (i,))
    rowbf = pl.BlockSpec((TILE, 3), lambda i: (i, 0))
    fullbf = pl.BlockSpec((3, p), lambda i: (0, 0))
    maskb = pl.BlockSpec((TILE, p), lambda i: (i, 0))

    nd, h, mask = pl.pallas_call(
        _stage1_kernel,
        grid=grid,
        in_specs=[full, full, rowb, rowbf, fullbf],
        out_specs=[rowb, vecb, maskb],
        out_shape=[
            jax.ShapeDtypeStruct((3, p), jnp.float32),
            jax.ShapeDtypeStruct((p,), jnp.float32),
            jax.ShapeDtypeStruct((p, p), jnp.int8),
        ],
        scratch_shapes=[
            pltpu.VMEM((TILE, p), jnp.float32),
            pltpu.VMEM((TILE, p), jnp.float32),
        ],
    )(pts, nrm, pts, pts_rows_bf, pts_all_bf)

    loss = pl.pallas_call(
        _stage2_kernel,
        grid=grid,
        in_specs=[full, full, rowb, rowb, vecb, maskb],
        out_specs=vecb,
        out_shape=jax.ShapeDtypeStruct((p,), jnp.float32),
    )(pts, nd, pts, nd, h, mask)

    return jnp.mean(loss)


def kernel(points, normals):
    return _run(points, normals)


# speedup vs baseline: 10.5233x; 2.2580x over previous
import jax
import jax.numpy as jnp
from jax.experimental import pallas as pl
from jax.experimental.pallas import tpu as pltpu

KNN_K = 33
SIGMA = 0.75
EPS = 1e-10
TILE = 128


def _eps_denom(x):
    return jnp.where(x < EPS, EPS, x)


def _noisy_d2(rowp_ref, rowbf_ref, allbf_ref, pts_ref):
    px = pts_ref[0, :][None, :]
    py = pts_ref[1, :][None, :]
    pz = pts_ref[2, :][None, :]
    rx = rowp_ref[0, :][:, None]
    ry = rowp_ref[1, :][:, None]
    rz = rowp_ref[2, :][:, None]
    sq_all = px * px + py * py + pz * pz
    sq_row = rx * rx + ry * ry + rz * rz
    cross = jax.lax.dot_general(
        rowbf_ref[...], allbf_ref[...],
        (((1,), (0,)), ((), ())),
        preferred_element_type=jnp.float32)
    return jnp.maximum(sq_row + sq_all - 2.0 * cross, 0.0)


def _exact_d2(rowp_ref, pts_ref):
    dx = rowp_ref[0, :][:, None] - pts_ref[0, :][None, :]
    dy = rowp_ref[1, :][:, None] - pts_ref[1, :][None, :]
    dz = rowp_ref[2, :][:, None] - pts_ref[2, :][None, :]
    return dx * dx + dy * dy + dz * dz


def _stage1_kernel(pts_ref, nrm_ref, rowp_ref, rowbf_ref, allbf_ref,
                   nd_ref, h_ref, mask_ref):
    d2 = _noisy_d2(rowp_ref, rowbf_ref, allbf_ref, pts_ref)
    t, p = d2.shape
    col = jax.lax.broadcasted_iota(jnp.int32, (t, p), 1)

    bits = jax.lax.bitcast_convert_type(d2, jnp.int32)

    def radix_body(b, pbits):
        cand = pbits + jax.lax.shift_left(jnp.int32(1), 30 - b)
        cnt = jnp.sum(jnp.where(bits < cand, 1.0, 0.0), axis=1,
                      keepdims=True)
        return jnp.where(cnt >= float(KNN_K), pbits, cand)

    pbits = jax.lax.fori_loop(
        0, 31, radix_body, jnp.zeros((t, 1), jnp.int32))
    v_b = jax.lax.bitcast_convert_type(pbits, jnp.float32)

    lt = d2 < v_b
    n_lt = jnp.sum(jnp.where(lt, 1.0, 0.0), axis=1, keepdims=True)
    tie = d2 == v_b
    t_need = float(KNN_K) - n_lt

    def idx_body(b, qbits):
        cand = qbits + jax.lax.shift_left(jnp.int32(1), 12 - b)
        cnt = jnp.sum(jnp.where(tie & (col < cand), 1.0, 0.0), axis=1,
                      keepdims=True)
        return jnp.where(cnt >= t_need, qbits, cand)

    ithr = jax.lax.fori_loop(
        0, 13, idx_body, jnp.zeros((t, 1), jnp.int32))
    kept33 = lt | (tie & (col <= ithr))
    m0 = jnp.min(d2, axis=1, keepdims=True)
    idrop = jnp.min(jnp.where(d2 == m0, col, p), axis=1, keepdims=True)
    kept = kept33 & (col != idrop)

    d = _exact_d2(rowp_ref, pts_ref)
    dm = jnp.where(kept, d, 0.0)
    h = jnp.sum(dm, axis=1, keepdims=True) * (4.0 / (KNN_K - 1.0))
    w = jnp.maximum(1.0 - d / _eps_denom(h), 0.0)
    w = w * w
    phi = jnp.where(kept, w * w, 0.0)
    den = _eps_denom(jnp.sum(phi, axis=1, keepdims=True))
    nx = nrm_ref[0, :][None, :]
    ny = nrm_ref[1, :][None, :]
    nz = nrm_ref[2, :][None, :]
    nd_ref[0, :] = jnp.sum(phi * nx, axis=1) / den[:, 0]
    nd_ref[1, :] = jnp.sum(phi * ny, axis=1) / den[:, 0]
    nd_ref[2, :] = jnp.sum(phi * nz, axis=1) / den[:, 0]
    h_ref[:] = h[:, 0]
    mask_ref[...] = kept.astype(jnp.int8)


def _stage2_kernel(pts_ref, nd_ref, rowp_ref, rownd_ref, h_ref, mask_ref,
                   loss_ref):
    kept = mask_ref[...] != 0
    d = _exact_d2(rowp_ref, pts_ref)
    h = h_ref[:][:, None]
    w = jnp.maximum(1.0 - d / _eps_denom(h), 0.0)
    w = w * w
    phi = jnp.where(kept, w * w, 0.0)

    ndx = nd_ref[0, :][None, :]
    ndy = nd_ref[1, :][None, :]
    ndz = nd_ref[2, :][None, :]
    inv_all = 1.0 / jnp.maximum(jnp.sqrt(ndx * ndx + ndy * ndy + ndz * ndz),
                                1e-12)
    ux, uy, uz = ndx * inv_all, ndy * inv_all, ndz * inv_all
    s_all = ux * ux + uy * uy + uz * uz

    rdx = rownd_ref[0, :][:, None]
    rdy = rownd_ref[1, :][:, None]
    rdz = rownd_ref[2, :][:, None]
    inv_row = 1.0 / jnp.maximum(jnp.sqrt(rdx * rdx + rdy * rdy + rdz * rdz),
                                1e-12)
    vx, vy, vz = rdx * inv_row, rdy * inv_row, rdz * inv_row
    s_row = vx * vx + vy * vy + vz * vz

    dot = vx * ux + vy * uy + vz * uz
    inv_sig = 1.0 / (SIGMA * SIGMA)
    normal_w = jnp.exp(-(s_row + s_all - 2.0 * dot) * inv_sig)
    w2 = phi * normal_w

    px = pts_ref[0, :][None, :]
    py = pts_ref[1, :][None, :]
    pz = pts_ref[2, :][None, :]
    rx = rowp_ref[0, :][:, None]
    ry = rowp_ref[1, :][:, None]
    rz = rowp_ref[2, :][:, None]
    a_all = px * ndx + py * ndy + pz * ndz
    pdot = rx * ndx + ry * ndy + rz * ndz
    inner = pdot - a_all

    num = jnp.sum(inner * w2, axis=1)
    den = _eps_denom(jnp.sum(w2, axis=1))
    dist = num / den
    loss_ref[:] = dist * dist


@jax.jit
def _run(points, normals):
    pts = points[0].T.astype(jnp.float32)
    nrm = normals[0].T.astype(jnp.float32)
    pts_rows_bf = points[0].astype(jnp.bfloat16)
    pts_all_bf = pts.astype(jnp.bfloat16)
    p = pts.shape[1]
    grid = (p // TILE,)
    full = pl.BlockSpec((3, p), lambda i: (0, 0))
    rowb = pl.BlockSpec((3, TILE), lambda i: (0, i))
    vecb = pl.BlockSpec((TILE,), lambda i: (i,))
    rowbf = pl.BlockSpec((TILE, 3), lambda i: (i, 0))
    fullbf = pl.BlockSpec((3, p), lambda i: (0, 0))
    maskb = pl.BlockSpec((TILE, p), lambda i: (i, 0))

    nd, h, mask = pl.pallas_call(
        _stage1_kernel,
        grid=grid,
        in_specs=[full, full, rowb, rowbf, fullbf],
        out_specs=[rowb, vecb, maskb],
        out_shape=[
            jax.ShapeDtypeStruct((3, p), jnp.float32),
            jax.ShapeDtypeStruct((p,), jnp.float32),
            jax.ShapeDtypeStruct((p, p), jnp.int8),
        ],
    )(pts, nrm, pts, pts_rows_bf, pts_all_bf)

    loss = pl.pallas_call(
        _stage2_kernel,
        grid=grid,
        in_specs=[full, full, rowb, rowb, vecb, maskb],
        out_specs=vecb,
        out_shape=jax.ShapeDtypeStruct((p,), jnp.float32),
    )(pts, nd, pts, nd, h, mask)

    return jnp.mean(loss)


def kernel(points, normals):
    return _run(points, normals)


# speedup vs baseline: 13.7888x; 1.3103x over previous
import jax
import jax.numpy as jnp
from jax.experimental import pallas as pl
from jax.experimental.pallas import tpu as pltpu

KNN_K = 33
SIGMA = 0.75
EPS = 1e-10
TILE = 128


def _eps_denom(x):
    return jnp.where(x < EPS, EPS, x)


def _noisy_d2(rowp_ref, rowbf_ref, allbf_ref, pts_ref):
    px = pts_ref[0, :][None, :]
    py = pts_ref[1, :][None, :]
    pz = pts_ref[2, :][None, :]
    rx = rowp_ref[0, :][:, None]
    ry = rowp_ref[1, :][:, None]
    rz = rowp_ref[2, :][:, None]
    sq_all = px * px + py * py + pz * pz
    sq_row = rx * rx + ry * ry + rz * rz
    cross = jax.lax.dot_general(
        rowbf_ref[...], allbf_ref[...],
        (((1,), (0,)), ((), ())),
        preferred_element_type=jnp.float32)
    return jnp.maximum(sq_row + sq_all - 2.0 * cross, 0.0)


def _exact_d2(rowp_ref, pts_ref):
    dx = rowp_ref[0, :][:, None] - pts_ref[0, :][None, :]
    dy = rowp_ref[1, :][:, None] - pts_ref[1, :][None, :]
    dz = rowp_ref[2, :][:, None] - pts_ref[2, :][None, :]
    return dx * dx + dy * dy + dz * dz


def _stage1_kernel(pts_ref, nrm_ref, rowp_ref, rowbf_ref, allbf_ref,
                   nd_ref, h_ref, mask_ref, cur_ref, kept_ref):
    d2 = _noisy_d2(rowp_ref, rowbf_ref, allbf_ref, pts_ref)
    t, p = d2.shape
    col = jax.lax.broadcasted_iota(jnp.int32, (t, p), 1)

    bits = jax.lax.bitcast_convert_type(d2, jnp.int32)

    def radix_body(b, pbits):
        cand = pbits + jax.lax.shift_left(jnp.int32(1), 30 - b)
        cnt = jnp.sum(jnp.where(bits < cand, 1.0, 0.0), axis=1,
                      keepdims=True)
        return jnp.where(cnt >= float(KNN_K), pbits, cand)

    RADIX_BITS = 19
    pbits = jax.lax.fori_loop(
        0, RADIX_BITS, radix_body, jnp.zeros((t, 1), jnp.int32))

    below = bits < pbits
    n_below = jnp.sum(jnp.where(below, 1.0, 0.0), axis=1, keepdims=True)
    kept_ref[...] = jnp.where(below, 1.0, 0.0)
    cur_ref[...] = jnp.where(below, jnp.inf, d2)

    def fill_cond(needed):
        return jnp.sum(needed) > 0.0

    def fill_body(needed):
        cur = cur_ref[...]
        m = jnp.min(cur, axis=1, keepdims=True)
        tie = cur == m
        isel = jnp.min(jnp.where(tie, col, p), axis=1, keepdims=True)
        rem = tie & (col == isel) & (needed > 0.0)
        cur_ref[...] = jnp.where(rem, jnp.inf, cur)
        kept_ref[...] = jnp.where(rem, 1.0, kept_ref[...])
        return needed - jnp.where(needed > 0.0, 1.0, 0.0)

    jax.lax.while_loop(fill_cond, fill_body, float(KNN_K) - n_below)

    kept33 = kept_ref[...] != 0.0
    m0 = jnp.min(d2, axis=1, keepdims=True)
    idrop = jnp.min(jnp.where(d2 == m0, col, p), axis=1, keepdims=True)
    kept = kept33 & (col != idrop)

    d = _exact_d2(rowp_ref, pts_ref)
    dm = jnp.where(kept, d, 0.0)
    h = jnp.sum(dm, axis=1, keepdims=True) * (4.0 / (KNN_K - 1.0))
    w = jnp.maximum(1.0 - d / _eps_denom(h), 0.0)
    w = w * w
    phi = jnp.where(kept, w * w, 0.0)
    den = _eps_denom(jnp.sum(phi, axis=1, keepdims=True))
    nx = nrm_ref[0, :][None, :]
    ny = nrm_ref[1, :][None, :]
    nz = nrm_ref[2, :][None, :]
    nd_ref[0, :] = jnp.sum(phi * nx, axis=1) / den[:, 0]
    nd_ref[1, :] = jnp.sum(phi * ny, axis=1) / den[:, 0]
    nd_ref[2, :] = jnp.sum(phi * nz, axis=1) / den[:, 0]
    h_ref[:] = h[:, 0]
    mask_ref[...] = kept.astype(jnp.int8)


def _stage2_kernel(pts_ref, nd_ref, rowp_ref, rownd_ref, h_ref, mask_ref,
                   loss_ref):
    kept = mask_ref[...] != 0
    d = _exact_d2(rowp_ref, pts_ref)
    h = h_ref[:][:, None]
    w = jnp.maximum(1.0 - d / _eps_denom(h), 0.0)
    w = w * w
    phi = jnp.where(kept, w * w, 0.0)

    ndx = nd_ref[0, :][None, :]
    ndy = nd_ref[1, :][None, :]
    ndz = nd_ref[2, :][None, :]
    inv_all = 1.0 / jnp.maximum(jnp.sqrt(ndx * ndx + ndy * ndy + ndz * ndz),
                                1e-12)
    ux, uy, uz = ndx * inv_all, ndy * inv_all, ndz * inv_all
    s_all = ux * ux + uy * uy + uz * uz

    rdx = rownd_ref[0, :][:, None]
    rdy = rownd_ref[1, :][:, None]
    rdz = rownd_ref[2, :][:, None]
    inv_row = 1.0 / jnp.maximum(jnp.sqrt(rdx * rdx + rdy * rdy + rdz * rdz),
                                1e-12)
    vx, vy, vz = rdx * inv_row, rdy * inv_row, rdz * inv_row
    s_row = vx * vx + vy * vy + vz * vz

    dot = vx * ux + vy * uy + vz * uz
    inv_sig = 1.0 / (SIGMA * SIGMA)
    normal_w = jnp.exp(-(s_row + s_all - 2.0 * dot) * inv_sig)
    w2 = phi * normal_w

    px = pts_ref[0, :][None, :]
    py = pts_ref[1, :][None, :]
    pz = pts_ref[2, :][None, :]
    rx = rowp_ref[0, :][:, None]
    ry = rowp_ref[1, :][:, None]
    rz = rowp_ref[2, :][:, None]
    a_all = px * ndx + py * ndy + pz * ndz
    pdot = rx * ndx + ry * ndy + rz * ndz
    inner = pdot - a_all

    num = jnp.sum(inner * w2, axis=1)
    den = _eps_denom(jnp.sum(w2, axis=1))
    dist = num / den
    loss_ref[:] = dist * dist


@jax.jit
def _run(points, normals):
    pts = points[0].T.astype(jnp.float32)
    nrm = normals[0].T.astype(jnp.float32)
    pts_rows_bf = points[0].astype(jnp.bfloat16)
    pts_all_bf = pts.astype(jnp.bfloat16)
    p = pts.shape[1]
    grid = (p // TILE,)
    full = pl.BlockSpec((3, p), lambda i: (0, 0))
    rowb = pl.BlockSpec((3, TILE), lambda i: (0, i))
    vecb = pl.BlockSpec((TILE,), lambda i: (i,))
    rowbf = pl.BlockSpec((TILE, 3), lambda i: (i, 0))
    fullbf = pl.BlockSpec((3, p), lambda i: (0, 0))
    maskb = pl.BlockSpec((TILE, p), lambda i: (i, 0))

    nd, h, mask = pl.pallas_call(
        _stage1_kernel,
        grid=grid,
        in_specs=[full, full, rowb, rowbf, fullbf],
        out_specs=[rowb, vecb, maskb],
        out_shape=[
            jax.ShapeDtypeStruct((3, p), jnp.float32),
            jax.ShapeDtypeStruct((p,), jnp.float32),
            jax.ShapeDtypeStruct((p, p), jnp.int8),
        ],
        scratch_shapes=[
            pltpu.VMEM((TILE, p), jnp.float32),
            pltpu.VMEM((TILE, p), jnp.float32),
        ],
    )(pts, nrm, pts, pts_rows_bf, pts_all_bf)

    loss = pl.pallas_call(
        _stage2_kernel,
        grid=grid,
        in_specs=[full, full, rowb, rowb, vecb, maskb],
        out_specs=vecb,
        out_shape=jax.ShapeDtypeStruct((p,), jnp.float32),
    )(pts, nd, pts, nd, h, mask)

    return jnp.mean(loss)


def kernel(points, normals):
    return _run(points, normals)
